# baseline (speedup 1.0000x reference)
import jax
import jax.numpy as jnp
from jax import lax
from jax.experimental import pallas as pl
from jax.experimental.pallas import tpu as pltpu
from jax.experimental.pallas import tpu_sc as plsc

N_NODES = 10000
N_EDGES = 320000
D = 64
NC, NS = 2, 16
NW = NC * NS
EPW = N_EDGES // NW
SUB = 50
NROW = 8
CHUNK = SUB * NROW
CHUNKS = EPW // CHUNK
EROWS = N_EDGES // SUB
ERPT = EROWS // NS
NP = 10240
NPW = NP // NS
HALF = NPW // 2


def _layer_body(xt_hbm, src_hbm, dst_hbm, ew2_hbm, ewf_hbm, out_hbm,
                y_sh, out_sh, sums_sh, sidx, didx, ewb, ewb2, rows,
                sums_buf, sem, sem2):
    core = lax.axis_index("c")
    sub = lax.axis_index("s")
    wid = core * NS + sub

    pltpu.sync_copy(xt_hbm.at[pl.ds(sub * NPW, NPW)],
                    y_sh.at[pl.ds(sub * NPW, NPW)])
    zero = jnp.zeros((16,), jnp.float32)

    def zbody(i, _):
        for j in range(4):
            rows[i, pl.ds(16 * j, 16)] = zero
        return 0

    lax.fori_loop(0, HALF, zbody, 0)
    pltpu.sync_copy(rows.at[pl.ds(0, HALF)],
                    out_sh.at[pl.ds(sub * NPW, HALF)])
    pltpu.sync_copy(rows.at[pl.ds(0, HALF)],
                    out_sh.at[pl.ds(sub * NPW + HALF, HALF)])

    def zsbody(i, _):
        sums_buf[pl.ds(i * 16, 16)] = zero
        return 0

    lax.fori_loop(0, NPW // 16, zsbody, 0)
    pltpu.sync_copy(sums_buf, sums_sh.at[pl.ds(sub * NPW, NPW)])
    plsc.subcore_barrier()

    def abody(k, _):
        base = sub * ERPT + k * NROW
        pltpu.sync_copy(src_hbm.at[pl.ds(base, NROW)], sidx)
        pltpu.sync_copy(ew2_hbm.at[pl.ds(base, NROW)], ewb2)
        descs = [
            pltpu.async_copy(ewb2.at[j], sums_sh.at[sidx.at[j]], sem2,
                             add=True)
            for j in range(NROW)
        ]
        for d_ in descs:
            d_.wait()
        return 0

    lax.fori_loop(0, ERPT // NROW, abody, 0)
    plsc.subcore_barrier()

    for piece in range(2):
        off = sub * NPW + piece * HALF
        pltpu.sync_copy(y_sh.at[pl.ds(off, HALF)], rows.at[pl.ds(0, HALF)])
        pltpu.sync_copy(sums_sh.at[pl.ds(off, HALF)],
                        sums_buf.at[pl.ds(0, HALF)])

        def dbody(g, _):
            sv = sums_buf[pl.ds(g * 16, 16)]
            rv = 1.0 / sv
            for j in range(16):
                s = rv[j]
                e = g * 16 + j
                for q in range(4):
                    rows[e, pl.ds(16 * q, 16)] = rows[e, pl.ds(16 * q, 16)] * s
            return 0

        lax.fori_loop(0, HALF // 16, dbody, 0)
        pltpu.sync_copy(rows.at[pl.ds(0, HALF)], y_sh.at[pl.ds(off, HALF)])
    plsc.subcore_barrier()

    def chunk_body(k, _):
        base = wid * (EPW // SUB) + k * NROW
        pltpu.sync_copy(src_hbm.at[pl.ds(base, NROW)], sidx)
        pltpu.sync_copy(dst_hbm.at[pl.ds(base, NROW)], didx)
        pltpu.sync_copy(ewf_hbm.at[pl.ds(wid * EPW + k * CHUNK, CHUNK)], ewb)
        descs = [
            pltpu.async_copy(y_sh.at[sidx.at[j]],
                             rows.at[pl.ds(j * SUB, SUB)], sem)
            for j in range(NROW)
        ]
        for d_ in descs:
            d_.wait()

        def gbody(g, _):
            ew16 = ewb[pl.ds(g * 16, 16)]
            for j in range(16):
                s = ew16[j]
                e = g * 16 + j
                for q in range(4):
                    rows[e, pl.ds(16 * q, 16)] = rows[e, pl.ds(16 * q, 16)] * s
            return 0

        lax.fori_loop(0, CHUNK // 16, gbody, 0)
        descs = [
            pltpu.async_copy(rows.at[pl.ds(j * SUB, SUB)],
                             out_sh.at[didx.at[j]], sem2, add=True)
            for j in range(NROW)
        ]
        for d_ in descs:
            d_.wait()
        return 0

    lax.fori_loop(0, CHUNKS, chunk_body, 0)

    plsc.subcore_barrier()
    pltpu.sync_copy(out_sh.at[pl.ds(sub * NPW, NPW)],
                    out_hbm.at[pl.ds(core * NP + sub * NPW, NPW)])


def _gnn_sparse(xt, src2, dst2, ew2, ewflat):
    xtp = jnp.concatenate(
        [xt, jnp.zeros((NP - N_NODES, D), jnp.float32)], axis=0)
    mesh = plsc.VectorSubcoreMesh(core_axis_name="c", subcore_axis_name="s")
    f = pl.kernel(
        _layer_body,
        out_type=jax.ShapeDtypeStruct((2 * NP, D), jnp.float32),
        mesh=mesh,
        compiler_params=pltpu.CompilerParams(use_tc_tiling_on_sc=False),
        scratch_types=[
            pltpu.VMEM_SHARED((NP, D), jnp.float32),
            pltpu.VMEM_SHARED((NP, D), jnp.float32),
            pltpu.VMEM_SHARED((NP,), jnp.float32),
            pltpu.VMEM((NROW, SUB), jnp.int32),
            pltpu.VMEM((NROW, SUB), jnp.int32),
            pltpu.VMEM((CHUNK,), jnp.float32),
            pltpu.VMEM((NROW, SUB), jnp.float32),
            pltpu.VMEM((CHUNK, D), jnp.float32),
            pltpu.VMEM((NPW,), jnp.float32),
            pltpu.SemaphoreType.DMA,
            pltpu.SemaphoreType.DMA,
        ],
    )
    out2 = f(xtp, src2, dst2, ew2, ewflat)
    return out2[:N_NODES] + out2[NP:NP + N_NODES]


def _lstm_scan(x_seq, p):
    n, t, d = x_seq.shape
    H = p['Whh'].shape[0]

    def step(carry, x_t):
        h, c = carry
        gates = x_t @ p['Wih'] + h @ p['Whh'] + p['bih'] + p['bhh']
        i, f, g, o = jnp.split(gates, 4, axis=-1)
        i = jax.nn.sigmoid(i)
        f = jax.nn.sigmoid(f)
        g = jnp.tanh(g)
        o = jax.nn.sigmoid(o)
        c = f * c + i * g
        h = o * jnp.tanh(c)
        return (h, c), h

    h0 = jnp.zeros((n, H), dtype=x_seq.dtype)
    c0 = jnp.zeros((n, H), dtype=x_seq.dtype)
    xs = jnp.swapaxes(x_seq, 0, 1)
    _, hs = jax.lax.scan(step, (h0, c0), xs)
    return jnp.swapaxes(hs, 0, 1)


def _node_transform(x, meta8, p):
    din = x.shape[1]
    Wafull = jnp.concatenate([p['Wa'], p['ba'][None, :]], axis=0)
    W3 = Wafull.reshape(8, din, din).transpose(2, 0, 1).reshape(din, 8 * din)
    T = (x @ W3).reshape(-1, 8, din)
    result = jnp.einsum('nk,nki->ni', meta8, T)
    beta = meta8[:, :7] @ p['Wb'] + p['bb']
    return result @ p['Wf'] + beta


def _out_matmul_body(x_ref, w_ref, b_ref, o_ref):
    o_ref[...] = x_ref[...] @ w_ref[...] + b_ref[...]


def kernel(x_sample, temporal_do, edge_index, edge_attr, area_id, params):
    n_nodes = x_sample.shape[0]
    mlp = params['mlp']
    h = jnp.maximum(x_sample @ mlp['W1'] + mlp['b1'], 0.0)
    sample_feature = h @ mlp['W2'] + mlp['b2']
    seq = temporal_do
    for l in range(2):
        pf = params['lstm'][2 * l]
        pb = params['lstm'][2 * l + 1]
        fwd = _lstm_scan(seq, pf)
        bwd = _lstm_scan(seq[:, ::-1, :], pb)[:, ::-1, :]
        seq = jnp.concatenate([fwd, bwd], axis=-1)
    temporal_feature = seq[:, 5, :]
    gnn_input = jnp.concatenate([sample_feature, temporal_feature], axis=1)

    meta = jnp.concatenate([x_sample[:, 1:5], x_sample[:, -3:]], axis=1)
    meta8 = jnp.concatenate([meta, jnp.ones((n_nodes, 1), meta.dtype)], axis=1)
    src, dst = edge_index[0], edge_index[1]
    src2 = src.reshape(EROWS, SUB)
    dst2 = dst.reshape(EROWS, SUB)

    x = gnn_input
    for l in range(2):
        p = params['gnn'][l]
        xt = _node_transform(x, meta8, p)
        ew = jnp.exp(edge_attr @ p['We'] + p['be'])
        out = _gnn_sparse(xt, src2, dst2,
                          ew.reshape(EROWS, SUB), ew.reshape(N_EDGES))
        x = jnp.maximum(out + xt, 0.0)

    wout = params['Wout']
    bout = jnp.broadcast_to(params['bout'][None, :], (n_nodes, wout.shape[1]))
    return pl.pallas_call(
        _out_matmul_body,
        out_shape=jax.ShapeDtypeStruct((n_nodes, wout.shape[1]), x.dtype),
    )(x, wout, bout)


# speedup vs baseline: 1.1516x; 1.1516x over previous
import jax
import jax.numpy as jnp
from jax import lax
from jax.experimental import pallas as pl
from jax.experimental.pallas import tpu as pltpu
from jax.experimental.pallas import tpu_sc as plsc

N_NODES = 10000
N_EDGES = 320000
D = 64
NC, NS = 2, 16
NW = NC * NS
EPW = N_EDGES // NW
SUB = 50
NROW = 8
CHUNK = SUB * NROW
CHUNKS = EPW // CHUNK
EROWS = N_EDGES // SUB
ERPT = EROWS // NS
ANROW = 40
NP = 10240
NPW = NP // NS
HALF = NPW // 2


def _layer_body(xt_hbm, src_hbm, dst_hbm, ew2_hbm, ewf_hbm, out_hbm,
                y_sh, out_sh, sums_sh, sidx, didx, ewb, sidxa, ewba, rows,
                sums_buf, sem, sem2, sem3):
    core = lax.axis_index("c")
    sub = lax.axis_index("s")
    wid = core * NS + sub

    pltpu.sync_copy(xt_hbm.at[pl.ds(sub * NPW, NPW)],
                    y_sh.at[pl.ds(sub * NPW, NPW)])
    zero = jnp.zeros((16,), jnp.float32)

    def zbody(i, _):
        for j in range(4):
            rows[i, pl.ds(16 * j, 16)] = zero
        return 0

    lax.fori_loop(0, HALF, zbody, 0)
    pltpu.sync_copy(rows.at[pl.ds(0, HALF)],
                    out_sh.at[pl.ds(sub * NPW, HALF)])
    pltpu.sync_copy(rows.at[pl.ds(0, HALF)],
                    out_sh.at[pl.ds(sub * NPW + HALF, HALF)])

    def zsbody(i, _):
        sums_buf[pl.ds(i * 16, 16)] = zero
        return 0

    lax.fori_loop(0, NPW // 16, zsbody, 0)
    pltpu.sync_copy(sums_buf, sums_sh.at[pl.ds(sub * NPW, NPW)])
    plsc.subcore_barrier()

    def abody(k, _):
        base = sub * ERPT + k * ANROW
        pltpu.sync_copy(src_hbm.at[pl.ds(base, ANROW)], sidxa)
        pltpu.sync_copy(ew2_hbm.at[pl.ds(base, ANROW)], ewba)
        descs = [
            pltpu.async_copy(ewba.at[j], sums_sh.at[sidxa.at[j]], sem2,
                             add=True)
            for j in range(ANROW)
        ]
        for d_ in descs:
            d_.wait()
        return 0

    lax.fori_loop(0, ERPT // ANROW, abody, 0)
    plsc.subcore_barrier()

    for piece in range(2):
        off = sub * NPW + piece * HALF
        pltpu.sync_copy(y_sh.at[pl.ds(off, HALF)], rows.at[pl.ds(0, HALF)])
        pltpu.sync_copy(sums_sh.at[pl.ds(off, HALF)],
                        sums_buf.at[pl.ds(0, HALF)])

        def dbody(g, _):
            sv = sums_buf[pl.ds(g * 16, 16)]
            rv = 1.0 / sv
            for j in range(16):
                s = rv[j]
                e = g * 16 + j
                for q in range(4):
                    rows[e, pl.ds(16 * q, 16)] = rows[e, pl.ds(16 * q, 16)] * s
            return 0

        lax.fori_loop(0, HALF // 16, dbody, 0)
        pltpu.sync_copy(rows.at[pl.ds(0, HALF)], y_sh.at[pl.ds(off, HALF)])
    plsc.subcore_barrier()

    def fire_loads(k, b):
        base = wid * (EPW // SUB) + k * NROW
        pltpu.async_copy(src_hbm.at[pl.ds(base, NROW)], sidx.at[b], sem3)
        pltpu.async_copy(dst_hbm.at[pl.ds(base, NROW)], didx.at[b], sem3)
        pltpu.async_copy(ewf_hbm.at[pl.ds(wid * EPW + k * CHUNK, CHUNK)],
                         ewb.at[b], sem3)

    def wait_loads(b):
        pltpu.make_async_copy(src_hbm.at[pl.ds(0, NROW)], sidx.at[b],
                              sem3).wait()
        pltpu.make_async_copy(dst_hbm.at[pl.ds(0, NROW)], didx.at[b],
                              sem3).wait()
        pltpu.make_async_copy(ewf_hbm.at[pl.ds(0, CHUNK)], ewb.at[b],
                              sem3).wait()

    def do_chunk(b):
        descs = [
            pltpu.async_copy(y_sh.at[sidx.at[b, j]],
                             rows.at[pl.ds(j * SUB, SUB)], sem)
            for j in range(NROW)
        ]
        for d_ in descs:
            d_.wait()

        def gbody(g, _):
            ew16 = ewb[b, pl.ds(g * 16, 16)]
            for j in range(16):
                s = ew16[j]
                e = g * 16 + j
                for q in range(4):
                    rows[e, pl.ds(16 * q, 16)] = rows[e, pl.ds(16 * q, 16)] * s
            return 0

        lax.fori_loop(0, CHUNK // 16, gbody, 0)
        descs = [
            pltpu.async_copy(rows.at[pl.ds(j * SUB, SUB)],
                             out_sh.at[didx.at[b, j]], sem2, add=True)
            for j in range(NROW)
        ]
        for d_ in descs:
            d_.wait()

    fire_loads(0, 0)

    def chunk_pair(k2, _):
        for b in range(2):
            k = 2 * k2 + b

            @pl.when(k < CHUNKS)
            def _():
                wait_loads(b)

                @pl.when(k < CHUNKS - 1)
                def _():
                    fire_loads(k + 1, 1 - b)

                do_chunk(b)
        return 0

    lax.fori_loop(0, (CHUNKS + 1) // 2, chunk_pair, 0)

    plsc.subcore_barrier()
    pltpu.sync_copy(out_sh.at[pl.ds(sub * NPW, NPW)],
                    out_hbm.at[pl.ds(core * NP + sub * NPW, NPW)])


def _gnn_sparse(xt, src2, dst2, ew2, ewflat):
    xtp = jnp.concatenate(
        [xt, jnp.zeros((NP - N_NODES, D), jnp.float32)], axis=0)
    mesh = plsc.VectorSubcoreMesh(core_axis_name="c", subcore_axis_name="s")
    f = pl.kernel(
        _layer_body,
        out_type=jax.ShapeDtypeStruct((2 * NP, D), jnp.float32),
        mesh=mesh,
        compiler_params=pltpu.CompilerParams(use_tc_tiling_on_sc=False),
        scratch_types=[
            pltpu.VMEM_SHARED((NP, D), jnp.float32),
            pltpu.VMEM_SHARED((NP, D), jnp.float32),
            pltpu.VMEM_SHARED((NP,), jnp.float32),
            pltpu.VMEM((2, NROW, SUB), jnp.int32),
            pltpu.VMEM((2, NROW, SUB), jnp.int32),
            pltpu.VMEM((2, CHUNK), jnp.float32),
            pltpu.VMEM((ANROW, SUB), jnp.int32),
            pltpu.VMEM((ANROW, SUB), jnp.float32),
            pltpu.VMEM((CHUNK, D), jnp.float32),
            pltpu.VMEM((NPW,), jnp.float32),
            pltpu.SemaphoreType.DMA,
            pltpu.SemaphoreType.DMA,
            pltpu.SemaphoreType.DMA,
        ],
    )
    out2 = f(xtp, src2, dst2, ew2, ewflat)
    return out2[:N_NODES] + out2[NP:NP + N_NODES]


def _lstm_scan(x_seq, p):
    n, t, d = x_seq.shape
    H = p['Whh'].shape[0]

    def step(carry, x_t):
        h, c = carry
        gates = x_t @ p['Wih'] + h @ p['Whh'] + p['bih'] + p['bhh']
        i, f, g, o = jnp.split(gates, 4, axis=-1)
        i = jax.nn.sigmoid(i)
        f = jax.nn.sigmoid(f)
        g = jnp.tanh(g)
        o = jax.nn.sigmoid(o)
        c = f * c + i * g
        h = o * jnp.tanh(c)
        return (h, c), h

    h0 = jnp.zeros((n, H), dtype=x_seq.dtype)
    c0 = jnp.zeros((n, H), dtype=x_seq.dtype)
    xs = jnp.swapaxes(x_seq, 0, 1)
    _, hs = jax.lax.scan(step, (h0, c0), xs)
    return jnp.swapaxes(hs, 0, 1)


def _node_transform(x, meta8, p):
    din = x.shape[1]
    Wafull = jnp.concatenate([p['Wa'], p['ba'][None, :]], axis=0)
    W3 = Wafull.reshape(8, din, din).transpose(2, 0, 1).reshape(din, 8 * din)
    T = (x @ W3).reshape(-1, 8, din)
    result = jnp.einsum('nk,nki->ni', meta8, T)
    beta = meta8[:, :7] @ p['Wb'] + p['bb']
    return result @ p['Wf'] + beta


def _out_matmul_body(x_ref, w_ref, b_ref, o_ref):
    o_ref[...] = x_ref[...] @ w_ref[...] + b_ref[...]


def kernel(x_sample, temporal_do, edge_index, edge_attr, area_id, params):
    n_nodes = x_sample.shape[0]
    mlp = params['mlp']
    h = jnp.maximum(x_sample @ mlp['W1'] + mlp['b1'], 0.0)
    sample_feature = h @ mlp['W2'] + mlp['b2']
    seq = temporal_do
    for l in range(2):
        pf = params['lstm'][2 * l]
        pb = params['lstm'][2 * l + 1]
        fwd = _lstm_scan(seq, pf)
        bwd = _lstm_scan(seq[:, ::-1, :], pb)[:, ::-1, :]
        seq = jnp.concatenate([fwd, bwd], axis=-1)
    temporal_feature = seq[:, 5, :]
    gnn_input = jnp.concatenate([sample_feature, temporal_feature], axis=1)

    meta = jnp.concatenate([x_sample[:, 1:5], x_sample[:, -3:]], axis=1)
    meta8 = jnp.concatenate([meta, jnp.ones((n_nodes, 1), meta.dtype)], axis=1)
    src, dst = edge_index[0], edge_index[1]
    src2 = src.reshape(EROWS, SUB)
    dst2 = dst.reshape(EROWS, SUB)

    x = gnn_input
    for l in range(2):
        p = params['gnn'][l]
        xt = _node_transform(x, meta8, p)
        ew = jnp.exp(edge_attr @ p['We'] + p['be'])
        out = _gnn_sparse(xt, src2, dst2,
                          ew.reshape(EROWS, SUB), ew.reshape(N_EDGES))
        x = jnp.maximum(out + xt, 0.0)

    wout = params['Wout']
    bout = jnp.broadcast_to(params['bout'][None, :], (n_nodes, wout.shape[1]))
    return pl.pallas_call(
        _out_matmul_body,
        out_shape=jax.ShapeDtypeStruct((n_nodes, wout.shape[1]), x.dtype),
    )(x, wout, bout)
